# Initial kernel scaffold; baseline (speedup 1.0000x reference)
import jax
import jax.numpy as jnp
from jax.experimental import pallas as pl


def kernel(seqs, species, vocab_emb, cat_emb, pos_emb):
    raise NotImplementedError("write your pallas kernel here")


# speedup vs baseline: 1.8925x; 1.8925x over previous
import jax
import jax.numpy as jnp
from jax.experimental import pallas as pl
from jax.experimental.pallas import tpu as pltpu

VPAD = 8


def _species_body(spe_idx_ref, cat_ref, out_ref):
    out_ref[...] = cat_ref[...]


def _total_body(seqs_ref, vocab_ref, spe_ref, pos_ref, out_ref):
    idx = seqs_ref[0, 0, :]
    blk = idx.shape[0]
    iota = jax.lax.broadcasted_iota(jnp.int32, (blk, VPAD), 1)
    oh = (idx[:, None] == iota).astype(jnp.float32)
    seq_emb = jnp.dot(oh, vocab_ref[...], preferred_element_type=jnp.float32)
    out_ref[...] = (seq_emb + spe_ref[0] + pos_ref[...])[None]


def kernel(seqs, species, vocab_emb, cat_emb, pos_emb):
    B, S = seqs.shape
    V, D = vocab_emb.shape
    blk = 512
    NB = S // blk

    seqs3 = seqs.astype(jnp.int32).reshape(B * NB, 1, blk)
    species32 = species.astype(jnp.int32)
    vocab_pad = jnp.concatenate(
        [vocab_emb, jnp.zeros((VPAD - V, D), vocab_emb.dtype)], axis=0
    )

    species_emb3 = pl.pallas_call(
        _species_body,
        grid_spec=pltpu.PrefetchScalarGridSpec(
            num_scalar_prefetch=1,
            grid=(B,),
            in_specs=[pl.BlockSpec((1, 1, D), lambda b, spe: (spe[b], 0, 0))],
            out_specs=pl.BlockSpec((1, 1, D), lambda b, spe: (b, 0, 0)),
        ),
        out_shape=jax.ShapeDtypeStruct((B, 1, D), jnp.float32),
    )(species32, cat_emb.reshape(cat_emb.shape[0], 1, D))
    species_emb = species_emb3.reshape(B, D)

    total = pl.pallas_call(
        _total_body,
        grid=(NB, B),
        in_specs=[
            pl.BlockSpec((1, 1, blk), lambda j, b: (b * NB + j, 0, 0)),
            pl.BlockSpec((VPAD, D), lambda j, b: (0, 0)),
            pl.BlockSpec((1, 1, D), lambda j, b: (b, 0, 0)),
            pl.BlockSpec((blk, D), lambda j, b: (j, 0)),
        ],
        out_specs=pl.BlockSpec((1, blk, D), lambda j, b: (b, j, 0)),
        out_shape=jax.ShapeDtypeStruct((B, S, D), jnp.float32),
        compiler_params=pltpu.CompilerParams(
            dimension_semantics=("arbitrary", "arbitrary")
        ),
    )(seqs3, vocab_pad, species_emb3, pos_emb)

    return (total, species_emb)


# speedup vs baseline: 2.1924x; 1.1584x over previous
import jax
import jax.numpy as jnp
from jax.experimental import pallas as pl
from jax.experimental.pallas import tpu as pltpu

VPAD = 8


def _species_body(spe_idx_ref, cat_ref, out_ref):
    out_ref[...] = cat_ref[...]


def _total_body(seqs_ref, vocab_ref, spe_ref, pos_ref, out_ref):
    idx = seqs_ref[0, 0, :]
    blk = idx.shape[0]
    iota = jax.lax.broadcasted_iota(jnp.int32, (blk, VPAD), 1)
    oh = (idx[:, None] == iota).astype(jnp.float32)
    seq_emb = jnp.dot(oh, vocab_ref[...], preferred_element_type=jnp.float32)
    out_ref[...] = (seq_emb + spe_ref[0] + pos_ref[...])[None]


def kernel(seqs, species, vocab_emb, cat_emb, pos_emb):
    B, S = seqs.shape
    V, D = vocab_emb.shape
    blk = 1024
    NB = S // blk

    seqs3 = seqs.astype(jnp.int32).reshape(B * NB, 1, blk)
    species32 = species.astype(jnp.int32)
    vocab_pad = jnp.concatenate(
        [vocab_emb, jnp.zeros((VPAD - V, D), vocab_emb.dtype)], axis=0
    )

    species_emb3 = pl.pallas_call(
        _species_body,
        grid_spec=pltpu.PrefetchScalarGridSpec(
            num_scalar_prefetch=1,
            grid=(B,),
            in_specs=[pl.BlockSpec((1, 1, D), lambda b, spe: (spe[b], 0, 0))],
            out_specs=pl.BlockSpec((1, 1, D), lambda b, spe: (b, 0, 0)),
        ),
        out_shape=jax.ShapeDtypeStruct((B, 1, D), jnp.float32),
    )(species32, cat_emb.reshape(cat_emb.shape[0], 1, D))
    species_emb = species_emb3.reshape(B, D)

    total = pl.pallas_call(
        _total_body,
        grid=(NB, B),
        in_specs=[
            pl.BlockSpec((1, 1, blk), lambda j, b: (b * NB + j, 0, 0)),
            pl.BlockSpec((VPAD, D), lambda j, b: (0, 0)),
            pl.BlockSpec((1, 1, D), lambda j, b: (b, 0, 0)),
            pl.BlockSpec((blk, D), lambda j, b: (j, 0)),
        ],
        out_specs=pl.BlockSpec((1, blk, D), lambda j, b: (b, j, 0)),
        out_shape=jax.ShapeDtypeStruct((B, S, D), jnp.float32),
        compiler_params=pltpu.CompilerParams(
            dimension_semantics=("arbitrary", "arbitrary")
        ),
    )(seqs3, vocab_pad, species_emb3, pos_emb)

    return (total, species_emb)


# speedup vs baseline: 2.3380x; 1.0664x over previous
import jax
import jax.numpy as jnp
from jax.experimental import pallas as pl
from jax.experimental.pallas import tpu as pltpu

VPAD = 8


def _species_body(spe_idx_ref, cat_ref, out_ref):
    out_ref[...] = cat_ref[...]


def _total_body(seqs_ref, vocab_ref, spe_ref, pos_ref, out_ref):
    idx = seqs_ref[0, 0, :]
    blk = idx.shape[0]
    iota = jax.lax.broadcasted_iota(jnp.int32, (blk, VPAD), 1)
    oh = (idx[:, None] == iota).astype(jnp.float32)
    seq_emb = jnp.dot(oh, vocab_ref[...], preferred_element_type=jnp.float32)
    out_ref[...] = (seq_emb + spe_ref[0] + pos_ref[...])[None]


def kernel(seqs, species, vocab_emb, cat_emb, pos_emb):
    B, S = seqs.shape
    V, D = vocab_emb.shape
    blk = 2048
    NB = S // blk

    seqs3 = seqs.astype(jnp.int32).reshape(B * NB, 1, blk)
    species32 = species.astype(jnp.int32)
    vocab_pad = jnp.concatenate(
        [vocab_emb, jnp.zeros((VPAD - V, D), vocab_emb.dtype)], axis=0
    )

    species_emb3 = pl.pallas_call(
        _species_body,
        grid_spec=pltpu.PrefetchScalarGridSpec(
            num_scalar_prefetch=1,
            grid=(B,),
            in_specs=[pl.BlockSpec((1, 1, D), lambda b, spe: (spe[b], 0, 0))],
            out_specs=pl.BlockSpec((1, 1, D), lambda b, spe: (b, 0, 0)),
        ),
        out_shape=jax.ShapeDtypeStruct((B, 1, D), jnp.float32),
    )(species32, cat_emb.reshape(cat_emb.shape[0], 1, D))
    species_emb = species_emb3.reshape(B, D)

    total = pl.pallas_call(
        _total_body,
        grid=(NB, B),
        in_specs=[
            pl.BlockSpec((1, 1, blk), lambda j, b: (b * NB + j, 0, 0)),
            pl.BlockSpec((VPAD, D), lambda j, b: (0, 0)),
            pl.BlockSpec((1, 1, D), lambda j, b: (b, 0, 0)),
            pl.BlockSpec((blk, D), lambda j, b: (j, 0)),
        ],
        out_specs=pl.BlockSpec((1, blk, D), lambda j, b: (b, j, 0)),
        out_shape=jax.ShapeDtypeStruct((B, S, D), jnp.float32),
        compiler_params=pltpu.CompilerParams(
            dimension_semantics=("arbitrary", "arbitrary")
        ),
    )(seqs3, vocab_pad, species_emb3, pos_emb)

    return (total, species_emb)
